# baseline (speedup 1.0000x reference)
import jax
import jax.numpy as jnp
from jax import lax
from jax.experimental import pallas as pl
from jax.experimental.pallas import tpu as pltpu
from jax.experimental.pallas import tpu_sc as plsc

N_NODES = 1000000
NBX = NBY = 1024
PAIR_ROWS = NBX // 2
MAP_WORDS = PAIR_ROWS * NBY

NUM_CORES = 2
NUM_SUBCORES = 16
NW = NUM_CORES * NUM_SUBCORES
LANES = 16

B = 1120
NBLK = 28
C = B * NBLK
NP = NW * C

HI_MASK = -65536


def _lo(w):
    return lax.bitcast_convert_type(w << 16, jnp.float32)


def _hi(w):
    return lax.bitcast_convert_type(w & HI_MASK, jnp.float32)


def _body(xs, ys, sxs, sys_, pws, map_hbm, out_hbm, map_sp,
          xb, yb, sxb0, sxb1, syb0, syb1, pwb0, pwb1,
          idx0, idx1, u0b, u1b, oy0b, oy1b, val0, val1, out0, out1,
          sem0, sem1):
    cid = lax.axis_index("c")
    sid = lax.axis_index("s")
    wid = sid * NUM_CORES + cid

    @pl.when(sid == 0)
    def _():
        pltpu.sync_copy(map_hbm, map_sp)

    plsc.subcore_barrier()

    def gen(blk, sxb, syb, pwb, idxb, ub, oyb):
        base = wid * C + blk * B
        pltpu.sync_copy(xs.at[pl.ds(base, B)], xb)
        pltpu.sync_copy(ys.at[pl.ds(base, B)], yb)
        pltpu.sync_copy(sxs.at[pl.ds(base, B)], sxb)
        pltpu.sync_copy(sys_.at[pl.ds(base, B)], syb)
        pltpu.sync_copy(pws.at[pl.ds(base, B)], pwb)

        def body(c, _):
            o = c * LANES
            x = xb[pl.ds(o, LANES)]
            y = yb[pl.ds(o, LANES)]
            sx = sxb[pl.ds(o, LANES)]
            sy = syb[pl.ds(o, LANES)]
            x2 = x + sx
            y2 = y + sy
            bxl = x.astype(jnp.int32)
            byl = y.astype(jnp.int32)
            bxf = bxl.astype(jnp.float32)
            byf = byl.astype(jnp.float32)
            ox = [jnp.maximum(
                jnp.minimum(x2, bxf + (d + 1.0)) - jnp.maximum(x, bxf + float(d)),
                0.0) for d in range(3)]
            oy = [jnp.maximum(
                jnp.minimum(y2, byf + (d + 1.0)) - jnp.maximum(y, byf + float(d)),
                0.0) for d in range(3)]
            bxh = (x * 0.5).astype(jnp.int32)
            pb = bxh * NBY + byl
            odd = (bxl - 2 * bxh).astype(jnp.float32)
            even = 1.0 - odd
            u = [even * ox[0],
                 even * ox[1] + odd * ox[0],
                 even * ox[2] + odd * ox[1],
                 odd * ox[2]]
            for dy in range(3):
                for j in range(2):
                    k = dy * 2 + j
                    idxb[pl.ds(k * B + o, LANES)] = pb + (j * NBY + dy)
            for q in range(4):
                ub[pl.ds(q * B + o, LANES)] = u[q]
            for r in range(3):
                oyb[pl.ds(r * B + o, LANES)] = oy[r]
            return 0

        lax.fori_loop(0, B // LANES, body, 0)

    def acc(blk, sxb, syb, pwb, ub, oyb, valb, outb):
        def body(c, _):
            o = c * LANES
            uu = [ub[pl.ds(q * B + o, LANES)] for q in range(4)]
            s = jnp.zeros((LANES,), jnp.float32)
            for dy in range(3):
                w0 = valb[pl.ds((dy * 2) * B + o, LANES)]
                w1 = valb[pl.ds((dy * 2 + 1) * B + o, LANES)]
                inner = (_lo(w0) * uu[0] + _hi(w0) * uu[1]
                         + _lo(w1) * uu[2] + _hi(w1) * uu[3])
                s = s + oyb[pl.ds(dy * B + o, LANES)] * inner
            sx = sxb[pl.ds(o, LANES)]
            sy = syb[pl.ds(o, LANES)]
            pw = pwb[pl.ds(o, LANES)]
            outb[pl.ds(o, LANES)] = s * (10.0 * pw) / (sx * sy)
            return 0

        lax.fori_loop(0, B // LANES, body, 0)
        base = wid * C + blk * B
        pltpu.sync_copy(outb, out_hbm.at[pl.ds(base, B)])

    def step(t, _):
        b0 = 2 * t
        b1 = 2 * t + 1
        gen(b0, sxb0, syb0, pwb0, idx0, u0b, oy0b)
        d0 = pltpu.async_copy(map_sp.at[idx0], val0, sem0)

        @pl.when(t > 0)
        def _():
            pltpu.make_async_copy(map_sp.at[idx1], val1, sem1).wait()
            acc(b0 - 1, sxb1, syb1, pwb1, u1b, oy1b, val1, out1)

        gen(b1, sxb1, syb1, pwb1, idx1, u1b, oy1b)
        pltpu.async_copy(map_sp.at[idx1], val1, sem1)
        d0.wait()
        acc(b0, sxb0, syb0, pwb0, u0b, oy0b, val0, out0)
        return 0

    lax.fori_loop(0, NBLK // 2, step, 0)
    pltpu.make_async_copy(map_sp.at[idx1], val1, sem1).wait()
    acc(NBLK - 1, sxb1, syb1, pwb1, u1b, oy1b, val1, out1)


@jax.jit
def _run(xs, ys, sxs, sys_, pws, map_words):
    mesh = plsc.VectorSubcoreMesh(core_axis_name="c", subcore_axis_name="s")
    return pl.kernel(
        _body,
        out_type=jax.ShapeDtypeStruct((NP,), jnp.float32),
        mesh=mesh,
        scratch_types=[
            pltpu.VMEM_SHARED((MAP_WORDS,), jnp.int32),
            pltpu.VMEM((B,), jnp.float32),
            pltpu.VMEM((B,), jnp.float32),
            pltpu.VMEM((B,), jnp.float32),
            pltpu.VMEM((B,), jnp.float32),
            pltpu.VMEM((B,), jnp.float32),
            pltpu.VMEM((B,), jnp.float32),
            pltpu.VMEM((B,), jnp.float32),
            pltpu.VMEM((B,), jnp.float32),
            pltpu.VMEM((6 * B,), jnp.int32),
            pltpu.VMEM((6 * B,), jnp.int32),
            pltpu.VMEM((4 * B,), jnp.float32),
            pltpu.VMEM((4 * B,), jnp.float32),
            pltpu.VMEM((3 * B,), jnp.float32),
            pltpu.VMEM((3 * B,), jnp.float32),
            pltpu.VMEM((6 * B,), jnp.int32),
            pltpu.VMEM((6 * B,), jnp.int32),
            pltpu.VMEM((B,), jnp.float32),
            pltpu.VMEM((B,), jnp.float32),
            pltpu.SemaphoreType.DMA,
            pltpu.SemaphoreType.DMA,
        ],
    )(xs, ys, sxs, sys_, pws, map_words)


def kernel(pos, node_size_x, node_size_y, utilization_map, pin_weights):
    n = N_NODES
    pad = NP - n
    x = jnp.concatenate([pos[:n], jnp.zeros((pad,), jnp.float32)])
    y = jnp.concatenate([pos[n:2 * n], jnp.zeros((pad,), jnp.float32)])
    sx = jnp.concatenate([node_size_x[:n], jnp.ones((pad,), jnp.float32)])
    sy = jnp.concatenate([node_size_y[:n], jnp.ones((pad,), jnp.float32)])
    pw = jnp.concatenate([pin_weights[:n], jnp.zeros((pad,), jnp.float32)])
    b = lax.bitcast_convert_type(utilization_map, jnp.int32)
    r = b + 32767 + ((b >> 16) & 1)
    rlo = r[0::2, :]
    rhi = r[1::2, :]
    words = ((rlo >> 16) | (rhi & HI_MASK)).reshape(-1)
    out = _run(x, y, sx, sy, pw, words)
    return out[:n]


# speedup vs baseline: 1.0442x; 1.0442x over previous
import jax
import jax.numpy as jnp
from jax import lax
from jax.experimental import pallas as pl
from jax.experimental.pallas import tpu as pltpu
from jax.experimental.pallas import tpu_sc as plsc

N_NODES = 1000000
NBX = NBY = 1024
PAIR_ROWS = NBX // 2
MAP_WORDS = PAIR_ROWS * NBY

NUM_CORES = 2
NUM_SUBCORES = 16
NW = NUM_CORES * NUM_SUBCORES
LANES = 16

B = 1984
NBLK = 16
C = B * NBLK
NP = NW * C

HI_MASK = -65536


def _lo(w):
    return lax.bitcast_convert_type(w << 16, jnp.float32)


def _hi(w):
    return lax.bitcast_convert_type(w & HI_MASK, jnp.float32)


def _body(xs, ys, sxs, sys_, pws, map_hbm, out_hbm, map_sp,
          xb, yb, sxb0, sxb1, syb0, syb1, pwb0, pwb1,
          idx0, idx1, u0b, u1b, oy0b, oy1b, val0, val1, out0, out1,
          sem0, sem1):
    cid = lax.axis_index("c")
    sid = lax.axis_index("s")
    wid = sid * NUM_CORES + cid

    @pl.when(sid == 0)
    def _():
        pltpu.sync_copy(map_hbm, map_sp)

    plsc.subcore_barrier()

    def gen(blk, sxb, syb, pwb, idxb, ub, oyb):
        base = wid * C + blk * B
        pltpu.sync_copy(xs.at[pl.ds(base, B)], xb)
        pltpu.sync_copy(ys.at[pl.ds(base, B)], yb)
        pltpu.sync_copy(sxs.at[pl.ds(base, B)], sxb)
        pltpu.sync_copy(sys_.at[pl.ds(base, B)], syb)
        pltpu.sync_copy(pws.at[pl.ds(base, B)], pwb)

        def body(c, _):
            o = c * LANES
            x = xb[pl.ds(o, LANES)]
            y = yb[pl.ds(o, LANES)]
            sx = sxb[pl.ds(o, LANES)]
            sy = syb[pl.ds(o, LANES)]
            x2 = x + sx
            y2 = y + sy
            bxl = x.astype(jnp.int32)
            byl = y.astype(jnp.int32)
            bxf = bxl.astype(jnp.float32)
            byf = byl.astype(jnp.float32)
            ox = [jnp.maximum(
                jnp.minimum(x2, bxf + (d + 1.0)) - jnp.maximum(x, bxf + float(d)),
                0.0) for d in range(3)]
            oy = [jnp.maximum(
                jnp.minimum(y2, byf + (d + 1.0)) - jnp.maximum(y, byf + float(d)),
                0.0) for d in range(3)]
            bxh = (x * 0.5).astype(jnp.int32)
            pb = bxh * NBY + byl
            odd = (bxl - 2 * bxh).astype(jnp.float32)
            even = 1.0 - odd
            u = [even * ox[0],
                 even * ox[1] + odd * ox[0],
                 even * ox[2] + odd * ox[1],
                 odd * ox[2]]
            for dy in range(3):
                for j in range(2):
                    k = dy * 2 + j
                    idxb[pl.ds(k * B + o, LANES)] = pb + (j * NBY + dy)
            for q in range(4):
                ub[pl.ds(q * B + o, LANES)] = u[q]
            for r in range(3):
                oyb[pl.ds(r * B + o, LANES)] = oy[r]
            return 0

        lax.fori_loop(0, B // LANES, body, 0)

    def acc(blk, sxb, syb, pwb, ub, oyb, valb, outb):
        def body(c, _):
            o = c * LANES
            uu = [ub[pl.ds(q * B + o, LANES)] for q in range(4)]
            s = jnp.zeros((LANES,), jnp.float32)
            for dy in range(3):
                w0 = valb[pl.ds((dy * 2) * B + o, LANES)]
                w1 = valb[pl.ds((dy * 2 + 1) * B + o, LANES)]
                inner = (_lo(w0) * uu[0] + _hi(w0) * uu[1]
                         + _lo(w1) * uu[2] + _hi(w1) * uu[3])
                s = s + oyb[pl.ds(dy * B + o, LANES)] * inner
            sx = sxb[pl.ds(o, LANES)]
            sy = syb[pl.ds(o, LANES)]
            pw = pwb[pl.ds(o, LANES)]
            outb[pl.ds(o, LANES)] = s * (10.0 * pw) / (sx * sy)
            return 0

        lax.fori_loop(0, B // LANES, body, 0)
        base = wid * C + blk * B
        pltpu.sync_copy(outb, out_hbm.at[pl.ds(base, B)])

    def step(t, _):
        b0 = 2 * t
        b1 = 2 * t + 1
        gen(b0, sxb0, syb0, pwb0, idx0, u0b, oy0b)
        d0 = pltpu.async_copy(map_sp.at[idx0], val0, sem0)

        @pl.when(t > 0)
        def _():
            pltpu.make_async_copy(map_sp.at[idx1], val1, sem1).wait()
            acc(b0 - 1, sxb1, syb1, pwb1, u1b, oy1b, val1, out1)

        gen(b1, sxb1, syb1, pwb1, idx1, u1b, oy1b)
        pltpu.async_copy(map_sp.at[idx1], val1, sem1)
        d0.wait()
        acc(b0, sxb0, syb0, pwb0, u0b, oy0b, val0, out0)
        return 0

    lax.fori_loop(0, NBLK // 2, step, 0)
    pltpu.make_async_copy(map_sp.at[idx1], val1, sem1).wait()
    acc(NBLK - 1, sxb1, syb1, pwb1, u1b, oy1b, val1, out1)


@jax.jit
def _run(xs, ys, sxs, sys_, pws, map_words):
    mesh = plsc.VectorSubcoreMesh(core_axis_name="c", subcore_axis_name="s")
    return pl.kernel(
        _body,
        out_type=jax.ShapeDtypeStruct((NP,), jnp.float32),
        mesh=mesh,
        scratch_types=[
            pltpu.VMEM_SHARED((MAP_WORDS,), jnp.int32),
            pltpu.VMEM((B,), jnp.float32),
            pltpu.VMEM((B,), jnp.float32),
            pltpu.VMEM((B,), jnp.float32),
            pltpu.VMEM((B,), jnp.float32),
            pltpu.VMEM((B,), jnp.float32),
            pltpu.VMEM((B,), jnp.float32),
            pltpu.VMEM((B,), jnp.float32),
            pltpu.VMEM((B,), jnp.float32),
            pltpu.VMEM((6 * B,), jnp.int32),
            pltpu.VMEM((6 * B,), jnp.int32),
            pltpu.VMEM((4 * B,), jnp.float32),
            pltpu.VMEM((4 * B,), jnp.float32),
            pltpu.VMEM((3 * B,), jnp.float32),
            pltpu.VMEM((3 * B,), jnp.float32),
            pltpu.VMEM((6 * B,), jnp.int32),
            pltpu.VMEM((6 * B,), jnp.int32),
            pltpu.VMEM((B,), jnp.float32),
            pltpu.VMEM((B,), jnp.float32),
            pltpu.SemaphoreType.DMA,
            pltpu.SemaphoreType.DMA,
        ],
    )(xs, ys, sxs, sys_, pws, map_words)


def kernel(pos, node_size_x, node_size_y, utilization_map, pin_weights):
    n = N_NODES
    pad = NP - n
    x = jnp.concatenate([pos[:n], jnp.zeros((pad,), jnp.float32)])
    y = jnp.concatenate([pos[n:2 * n], jnp.zeros((pad,), jnp.float32)])
    sx = jnp.concatenate([node_size_x[:n], jnp.ones((pad,), jnp.float32)])
    sy = jnp.concatenate([node_size_y[:n], jnp.ones((pad,), jnp.float32)])
    pw = jnp.concatenate([pin_weights[:n], jnp.zeros((pad,), jnp.float32)])
    b = lax.bitcast_convert_type(utilization_map, jnp.int32)
    r = b + 32767 + ((b >> 16) & 1)
    rlo = r[0::2, :]
    rhi = r[1::2, :]
    words = ((rlo >> 16) | (rhi & HI_MASK)).reshape(-1)
    out = _run(x, y, sx, sy, pw, words)
    return out[:n]


# speedup vs baseline: 1.1128x; 1.0657x over previous
import jax
import jax.numpy as jnp
from jax import lax
from jax.experimental import pallas as pl
from jax.experimental.pallas import tpu as pltpu
from jax.experimental.pallas import tpu_sc as plsc

N_NODES = 1000000
NBX = NBY = 1024
PAIR_ROWS = NBX // 2
MAP_WORDS = PAIR_ROWS * NBY

NUM_CORES = 2
NUM_SUBCORES = 16
NW = NUM_CORES * NUM_SUBCORES
LANES = 16

B = 1568
NBLK = 20
C = B * NBLK
NP = NW * C

HI_MASK = -65536


def _lo(w):
    return lax.bitcast_convert_type(w << 16, jnp.float32)


def _hi(w):
    return lax.bitcast_convert_type(w & HI_MASK, jnp.float32)


def _body(xs, ys, sxs, sys_, pws, map_hbm, out_hbm, map_sp,
          xb, yb, sxb0, sxb1, syb0, syb1, pwb0, pwb1,
          idx0, idx1, u0b, u1b, oy0b, oy1b, val0, val1, out0, out1,
          sem0, sem1):
    cid = lax.axis_index("c")
    sid = lax.axis_index("s")
    wid = sid * NUM_CORES + cid

    @pl.when(sid == 0)
    def _():
        pltpu.sync_copy(map_hbm, map_sp)

    plsc.subcore_barrier()

    def gen(blk, sxb, syb, pwb, idxb, ub, oyb):
        base = wid * C + blk * B
        pltpu.sync_copy(xs.at[pl.ds(base, B)], xb)
        pltpu.sync_copy(ys.at[pl.ds(base, B)], yb)
        pltpu.sync_copy(sxs.at[pl.ds(base, B)], sxb)
        pltpu.sync_copy(sys_.at[pl.ds(base, B)], syb)
        pltpu.sync_copy(pws.at[pl.ds(base, B)], pwb)

        def body(c, _):
            o = c * LANES
            x = xb[pl.ds(o, LANES)]
            y = yb[pl.ds(o, LANES)]
            sx = sxb[pl.ds(o, LANES)]
            sy = syb[pl.ds(o, LANES)]
            x2 = x + sx
            y2 = y + sy
            bxl = x.astype(jnp.int32)
            byl = y.astype(jnp.int32)
            bxf = bxl.astype(jnp.float32)
            byf = byl.astype(jnp.float32)
            ox = [jnp.maximum(
                jnp.minimum(x2, bxf + (d + 1.0)) - jnp.maximum(x, bxf + float(d)),
                0.0) for d in range(3)]
            oy = [jnp.maximum(
                jnp.minimum(y2, byf + (d + 1.0)) - jnp.maximum(y, byf + float(d)),
                0.0) for d in range(3)]
            bxh = (x * 0.5).astype(jnp.int32)
            pb = bxh * NBY + byl
            odd = (bxl - 2 * bxh).astype(jnp.float32)
            even = 1.0 - odd
            u = [even * ox[0],
                 even * ox[1] + odd * ox[0],
                 even * ox[2] + odd * ox[1],
                 odd * ox[2]]
            for dy in range(3):
                for j in range(2):
                    k = dy * 2 + j
                    idxb[pl.ds(k * B + o, LANES)] = pb + (j * NBY + dy)
            for q in range(4):
                ub[pl.ds(q * B + o, LANES)] = u[q]
            for r in range(3):
                oyb[pl.ds(r * B + o, LANES)] = oy[r]
            return 0

        lax.fori_loop(0, B // LANES, body, 0)

    def acc(blk, sxb, syb, pwb, ub, oyb, valb, outb):
        def body(c, _):
            o = c * LANES
            uu = [ub[pl.ds(q * B + o, LANES)] for q in range(4)]
            s = jnp.zeros((LANES,), jnp.float32)
            for dy in range(3):
                w0 = valb[pl.ds((dy * 2) * B + o, LANES)]
                w1 = valb[pl.ds((dy * 2 + 1) * B + o, LANES)]
                inner = (_lo(w0) * uu[0] + _hi(w0) * uu[1]
                         + _lo(w1) * uu[2] + _hi(w1) * uu[3])
                s = s + oyb[pl.ds(dy * B + o, LANES)] * inner
            sx = sxb[pl.ds(o, LANES)]
            sy = syb[pl.ds(o, LANES)]
            pw = pwb[pl.ds(o, LANES)]
            outb[pl.ds(o, LANES)] = s * (10.0 * pw) / (sx * sy)
            return 0

        lax.fori_loop(0, B // LANES, body, 0)
        base = wid * C + blk * B
        pltpu.sync_copy(outb, out_hbm.at[pl.ds(base, B)])

    def step(t, _):
        b0 = 2 * t
        b1 = 2 * t + 1
        gen(b0, sxb0, syb0, pwb0, idx0, u0b, oy0b)
        d0 = pltpu.async_copy(map_sp.at[idx0], val0, sem0)

        @pl.when(t > 0)
        def _():
            pltpu.make_async_copy(map_sp.at[idx1], val1, sem1).wait()
            acc(b0 - 1, sxb1, syb1, pwb1, u1b, oy1b, val1, out1)

        gen(b1, sxb1, syb1, pwb1, idx1, u1b, oy1b)
        pltpu.async_copy(map_sp.at[idx1], val1, sem1)
        d0.wait()
        acc(b0, sxb0, syb0, pwb0, u0b, oy0b, val0, out0)
        return 0

    lax.fori_loop(0, NBLK // 2, step, 0)
    pltpu.make_async_copy(map_sp.at[idx1], val1, sem1).wait()
    acc(NBLK - 1, sxb1, syb1, pwb1, u1b, oy1b, val1, out1)


@jax.jit
def _run(xs, ys, sxs, sys_, pws, map_words):
    mesh = plsc.VectorSubcoreMesh(core_axis_name="c", subcore_axis_name="s")
    return pl.kernel(
        _body,
        out_type=jax.ShapeDtypeStruct((NP,), jnp.float32),
        mesh=mesh,
        scratch_types=[
            pltpu.VMEM_SHARED((MAP_WORDS,), jnp.int32),
            pltpu.VMEM((B,), jnp.float32),
            pltpu.VMEM((B,), jnp.float32),
            pltpu.VMEM((B,), jnp.float32),
            pltpu.VMEM((B,), jnp.float32),
            pltpu.VMEM((B,), jnp.float32),
            pltpu.VMEM((B,), jnp.float32),
            pltpu.VMEM((B,), jnp.float32),
            pltpu.VMEM((B,), jnp.float32),
            pltpu.VMEM((6 * B,), jnp.int32),
            pltpu.VMEM((6 * B,), jnp.int32),
            pltpu.VMEM((4 * B,), jnp.float32),
            pltpu.VMEM((4 * B,), jnp.float32),
            pltpu.VMEM((3 * B,), jnp.float32),
            pltpu.VMEM((3 * B,), jnp.float32),
            pltpu.VMEM((6 * B,), jnp.int32),
            pltpu.VMEM((6 * B,), jnp.int32),
            pltpu.VMEM((B,), jnp.float32),
            pltpu.VMEM((B,), jnp.float32),
            pltpu.SemaphoreType.DMA,
            pltpu.SemaphoreType.DMA,
        ],
    )(xs, ys, sxs, sys_, pws, map_words)


def kernel(pos, node_size_x, node_size_y, utilization_map, pin_weights):
    n = N_NODES
    pad = NP - n
    x = jnp.concatenate([pos[:n], jnp.zeros((pad,), jnp.float32)])
    y = jnp.concatenate([pos[n:2 * n], jnp.zeros((pad,), jnp.float32)])
    sx = jnp.concatenate([node_size_x[:n], jnp.ones((pad,), jnp.float32)])
    sy = jnp.concatenate([node_size_y[:n], jnp.ones((pad,), jnp.float32)])
    pw = jnp.concatenate([pin_weights[:n], jnp.zeros((pad,), jnp.float32)])
    b = lax.bitcast_convert_type(utilization_map, jnp.int32)
    r = b + 32767 + ((b >> 16) & 1)
    rlo = r[0::2, :]
    rhi = r[1::2, :]
    words = ((rlo >> 16) | (rhi & HI_MASK)).reshape(-1)
    out = _run(x, y, sx, sy, pw, words)
    return out[:n]
